# Initial kernel scaffold; baseline (speedup 1.0000x reference)
import jax
import jax.numpy as jnp
from jax.experimental import pallas as pl


def kernel(points, freqs, cv):
    raise NotImplementedError("write your pallas kernel here")


# speedup vs baseline: 106.2831x; 106.2831x over previous
import functools

import numpy as np
import jax
import jax.numpy as jnp
from jax import lax
from jax.experimental import pallas as pl
from jax.experimental.pallas import tpu as pltpu
from jax.experimental.pallas import tpu_sc as plsc

NUM_FREQS = 6
C = 16
RES = 16
NVOL = NUM_FREQS * 2 * 8
P = 2048
LANES = 16

_SIN_MASK = np.array([1.0 if ((r // 3) % 2) == 0 else 0.0
                      for r in range(NUM_FREQS * 2 * 3)],
                     dtype=np.float32).reshape(NUM_FREQS * 2 * 3, 1)


def _prep_body(p_ref, f_ref, m_ref, o_ref):
    fp = p_ref[...] * f_ref[...]
    m = m_ref[...]
    o_ref[...] = m * jnp.sin(fp) + (1.0 - m) * jnp.cos(fp)


def _tc_prep(pts36, fa36):
    R, N = pts36.shape
    BN = 4096
    return pl.pallas_call(
        _prep_body,
        grid=(N // BN,),
        in_specs=[
            pl.BlockSpec((R, BN), lambda i: (0, i)),
            pl.BlockSpec((R, 1), lambda i: (0, 0)),
            pl.BlockSpec((R, 1), lambda i: (0, 0)),
        ],
        out_specs=pl.BlockSpec((R, BN), lambda i: (0, i)),
        out_shape=jax.ShapeDtypeStruct((R, N), jnp.float32),
    )(pts36, fa36, jnp.asarray(_SIN_MASK))


def _axis_prep(g):
    p = (g + 1.0) * 0.5 * (RES - 1)
    p = jnp.minimum(jnp.maximum(p, 0.0), float(RES - 1))
    i0 = p.astype(jnp.int32)
    w = p - i0.astype(jnp.float32)
    i1 = jnp.minimum(i0 + 1, RES - 1)
    return i0, i1 - i0, w


def _sc_lookup(trig, cv_rl, n_points):
    N = n_points
    mesh = plsc.VectorSubcoreMesh(core_axis_name="c", subcore_axis_name="s")
    info = plsc.get_sparse_core_info()
    nw = info.num_cores * info.num_subcores
    vols_per_w = NVOL // nw
    n_chunks = N // P
    n_pvec = P // LANES

    @functools.partial(
        pl.kernel,
        mesh=mesh,
        out_type=jax.ShapeDtypeStruct((N, NVOL * C), jnp.float32),
        compiler_params=pltpu.CompilerParams(
            use_tc_tiling_on_sc=False, needs_layout_passes=False),
        scratch_types=[
            pltpu.VMEM((RES * RES * RES * C,), jnp.float32),
            pltpu.VMEM((P,), jnp.float32),
            pltpu.VMEM((P,), jnp.float32),
            pltpu.VMEM((P,), jnp.float32),
            pltpu.VMEM((P,), jnp.float32),
            pltpu.VMEM((P, C), jnp.float32),
        ],
    )
    def body(trig_hbm, cv_hbm, out_hbm, vol_v, gx_v, gy_v, gz_v, ad_v, ob_v):
        wid = lax.axis_index("s") * info.num_cores + lax.axis_index("c")
        for k in range(vols_per_w):
            b = wid + nw * k
            f = b // 16
            s = (b // 8) % 2
            cc = b % 8
            rx = (f * 2 + ((cc >> 2) & 1)) * 3 + 0
            ry = (f * 2 + ((cc >> 1) & 1)) * 3 + 1
            rz = (f * 2 + (cc & 1)) * 3 + 2
            ra = (f * 2 + s) * 3 + 0
            pltpu.sync_copy(cv_hbm.at[b], vol_v)

            def chunk_body(ci, _):
                n0 = ci * P
                pltpu.sync_copy(trig_hbm.at[rx, pl.ds(n0, P)], gx_v)
                pltpu.sync_copy(trig_hbm.at[ry, pl.ds(n0, P)], gy_v)
                pltpu.sync_copy(trig_hbm.at[rz, pl.ds(n0, P)], gz_v)
                pltpu.sync_copy(trig_hbm.at[ra, pl.ds(n0, P)], ad_v)

                def pvec_body(pv, __):
                    o = pv * LANES
                    ix0, dx, wx = _axis_prep(gx_v[pl.ds(o, LANES)])
                    iy0, dy, wy = _axis_prep(gy_v[pl.ds(o, LANES)])
                    iz0, dz, wz = _axis_prep(gz_v[pl.ds(o, LANES)])
                    ad = ad_v[pl.ds(o, LANES)]
                    base = (iz0 << 12) + (iy0 << 8) + (ix0 << 4)
                    dx16 = dx << 4
                    dy256 = dy << 8
                    dz4096 = dz << 12
                    i000 = base
                    i001 = base + dx16
                    i010 = base + dy256
                    i011 = i010 + dx16
                    i100 = base + dz4096
                    i101 = i100 + dx16
                    i110 = i100 + dy256
                    i111 = i110 + dx16
                    ux = 1.0 - wx
                    uy = 1.0 - wy
                    uz = 1.0 - wz
                    a00 = uy * ux
                    a01 = uy * wx
                    a10 = wy * ux
                    a11 = wy * wx
                    w000 = uz * a00
                    w001 = uz * a01
                    w010 = uz * a10
                    w011 = uz * a11
                    w100 = wz * a00
                    w101 = wz * a01
                    w110 = wz * a10
                    w111 = wz * a11
                    rows = lax.iota(jnp.int32, LANES) + o
                    for ch in range(C):
                        col = jnp.full((LANES,), ch, jnp.int32)
                        acc = ad
                        acc = acc + w000 * plsc.load_gather(vol_v, [i000 + ch])
                        acc = acc + w001 * plsc.load_gather(vol_v, [i001 + ch])
                        acc = acc + w010 * plsc.load_gather(vol_v, [i010 + ch])
                        acc = acc + w011 * plsc.load_gather(vol_v, [i011 + ch])
                        acc = acc + w100 * plsc.load_gather(vol_v, [i100 + ch])
                        acc = acc + w101 * plsc.load_gather(vol_v, [i101 + ch])
                        acc = acc + w110 * plsc.load_gather(vol_v, [i110 + ch])
                        acc = acc + w111 * plsc.load_gather(vol_v, [i111 + ch])
                        plsc.store_scatter(ob_v, [rows, col], acc)
                    return 0

                lax.fori_loop(0, n_pvec, pvec_body, 0)
                pltpu.sync_copy(
                    ob_v, out_hbm.at[pl.ds(n0, P), pl.ds(b * C, C)])
                return 0

            lax.fori_loop(0, n_chunks, chunk_body, 0)

    return body(trig, cv_rl)


def kernel(points, freqs, cv):
    N = points.shape[0]
    pts36 = jnp.tile(points.T, (NUM_FREQS * 2, 1))
    fa36 = jnp.repeat(freqs, 6)[:, None].astype(jnp.float32)
    trig = _tc_prep(pts36, fa36)
    cv_rl = jnp.transpose(cv, (0, 2, 3, 4, 1)).reshape(NVOL, RES * RES * RES * C)
    return _sc_lookup(trig, cv_rl, N)


# speedup vs baseline: 252.2427x; 2.3733x over previous
import functools

import numpy as np
import jax
import jax.numpy as jnp
from jax import lax
from jax.experimental import pallas as pl
from jax.experimental.pallas import tpu as pltpu
from jax.experimental.pallas import tpu_sc as plsc

NUM_FREQS = 6
C = 16
RES = 16
NVOL = NUM_FREQS * 2 * 8
P = 2048
LANES = 16

_SIN_MASK = np.array([1.0 if ((r // 3) % 2) == 0 else 0.0
                      for r in range(NUM_FREQS * 2 * 3)],
                     dtype=np.float32).reshape(NUM_FREQS * 2 * 3, 1)


def _prep_body(p_ref, f_ref, m_ref, o_ref):
    fp = p_ref[...] * f_ref[...]
    m = m_ref[...]
    o_ref[...] = m * jnp.sin(fp) + (1.0 - m) * jnp.cos(fp)


def _tc_prep(pts36, fa36):
    R, N = pts36.shape
    BN = 4096
    return pl.pallas_call(
        _prep_body,
        grid=(N // BN,),
        in_specs=[
            pl.BlockSpec((R, BN), lambda i: (0, i)),
            pl.BlockSpec((R, 1), lambda i: (0, 0)),
            pl.BlockSpec((R, 1), lambda i: (0, 0)),
        ],
        out_specs=pl.BlockSpec((R, BN), lambda i: (0, i)),
        out_shape=jax.ShapeDtypeStruct((R, N), jnp.float32),
    )(pts36, fa36, jnp.asarray(_SIN_MASK))


def _axis_prep(g):
    p = (g + 1.0) * 0.5 * (RES - 1)
    p = jnp.minimum(jnp.maximum(p, 0.0), float(RES - 1))
    i0 = p.astype(jnp.int32)
    w = p - i0.astype(jnp.float32)
    i1 = jnp.minimum(i0 + 1, RES - 1)
    return i0, i1 - i0, w


def _sc_lookup(trig, cv_rl, n_points):
    N = n_points
    mesh = plsc.VectorSubcoreMesh(core_axis_name="c", subcore_axis_name="s")
    info = plsc.get_sparse_core_info()
    nw = info.num_cores * info.num_subcores
    vols_per_w = NVOL // nw
    n_chunks = N // P
    n_pvec = P // LANES

    @functools.partial(
        pl.kernel,
        mesh=mesh,
        out_type=jax.ShapeDtypeStruct((N, NVOL * C), jnp.float32),
        compiler_params=pltpu.CompilerParams(
            use_tc_tiling_on_sc=False, needs_layout_passes=False),
        scratch_types=[
            pltpu.VMEM((RES * RES * RES * C,), jnp.float32),
            pltpu.VMEM((P,), jnp.float32),
            pltpu.VMEM((P,), jnp.float32),
            pltpu.VMEM((P,), jnp.float32),
            pltpu.VMEM((P,), jnp.float32),
            pltpu.VMEM((P, C), jnp.float32),
        ],
    )
    def body(trig_hbm, cv_hbm, out_hbm, vol_v, gx_v, gy_v, gz_v, ad_v, ob_v):
        wid = lax.axis_index("s") * info.num_cores + lax.axis_index("c")
        for k in range(vols_per_w):
            b = wid + nw * k
            f = b // 16
            s = (b // 8) % 2
            cc = b % 8
            rx = (f * 2 + ((cc >> 2) & 1)) * 3 + 0
            ry = (f * 2 + ((cc >> 1) & 1)) * 3 + 1
            rz = (f * 2 + (cc & 1)) * 3 + 2
            ra = (f * 2 + s) * 3 + 0
            pltpu.sync_copy(cv_hbm.at[b], vol_v)

            def chunk_body(ci, _):
                n0 = ci * P
                pltpu.sync_copy(trig_hbm.at[rx, pl.ds(n0, P)], gx_v)
                pltpu.sync_copy(trig_hbm.at[ry, pl.ds(n0, P)], gy_v)
                pltpu.sync_copy(trig_hbm.at[rz, pl.ds(n0, P)], gz_v)
                pltpu.sync_copy(trig_hbm.at[ra, pl.ds(n0, P)], ad_v)

                def pvec_body(pv, __):
                    o = pv * LANES
                    ix0, dx, wx = _axis_prep(gx_v[pl.ds(o, LANES)])
                    iy0, dy, wy = _axis_prep(gy_v[pl.ds(o, LANES)])
                    iz0, dz, wz = _axis_prep(gz_v[pl.ds(o, LANES)])
                    ad = ad_v[pl.ds(o, LANES)]
                    base = (iz0 << 12) + (iy0 << 8) + (ix0 << 4)
                    dx16 = dx << 4
                    dy256 = dy << 8
                    dz4096 = dz << 12
                    i000 = base
                    i001 = base + dx16
                    i010 = base + dy256
                    i011 = i010 + dx16
                    i100 = base + dz4096
                    i101 = i100 + dx16
                    i110 = i100 + dy256
                    i111 = i110 + dx16
                    ux = 1.0 - wx
                    uy = 1.0 - wy
                    uz = 1.0 - wz
                    a00 = uy * ux
                    a01 = uy * wx
                    a10 = wy * ux
                    a11 = wy * wx
                    w000 = uz * a00
                    w001 = uz * a01
                    w010 = uz * a10
                    w011 = uz * a11
                    w100 = wz * a00
                    w101 = wz * a01
                    w110 = wz * a10
                    w111 = wz * a11
                    lanes = lax.iota(jnp.int32, LANES)
                    rows = lanes + o
                    for ch in range(C):
                        col = (lanes + ch) & (C - 1)
                        acc = ad
                        acc = acc + w000 * plsc.load_gather(vol_v, [i000 + col])
                        acc = acc + w001 * plsc.load_gather(vol_v, [i001 + col])
                        acc = acc + w010 * plsc.load_gather(vol_v, [i010 + col])
                        acc = acc + w011 * plsc.load_gather(vol_v, [i011 + col])
                        acc = acc + w100 * plsc.load_gather(vol_v, [i100 + col])
                        acc = acc + w101 * plsc.load_gather(vol_v, [i101 + col])
                        acc = acc + w110 * plsc.load_gather(vol_v, [i110 + col])
                        acc = acc + w111 * plsc.load_gather(vol_v, [i111 + col])
                        plsc.store_scatter(ob_v, [rows, col], acc)
                    return 0

                lax.fori_loop(0, n_pvec, pvec_body, 0)
                pltpu.sync_copy(
                    ob_v, out_hbm.at[pl.ds(n0, P), pl.ds(b * C, C)])
                return 0

            lax.fori_loop(0, n_chunks, chunk_body, 0)

    return body(trig, cv_rl)


def kernel(points, freqs, cv):
    N = points.shape[0]
    pts36 = jnp.tile(points.T, (NUM_FREQS * 2, 1))
    fa36 = jnp.repeat(freqs, 6)[:, None].astype(jnp.float32)
    trig = _tc_prep(pts36, fa36)
    cv_rl = jnp.transpose(cv, (0, 2, 3, 4, 1)).reshape(NVOL, RES * RES * RES * C)
    return _sc_lookup(trig, cv_rl, N)
